# baseline (speedup 1.0000x reference)
import functools

import jax
import jax.numpy as jnp
from jax import lax
from jax.experimental import pallas as pl
from jax.experimental.pallas import tpu as pltpu
from jax.experimental.pallas import tpu_sc as plsc

_B, _S = 16, 20
_FD, _RD = 32, 32
_NF, _NR = 100000, 200
_Q = _B * _S
_QPAD = 512
_IDX_PER_W = _QPAD // 32
_CHUNK = 4000
_BIGF = 1e9
_UNROLL = 5


def _sc_gather_body(table_hbm, idx_hbm, out_hbm, idx_v, rows_v, sem):
    wid = lax.axis_index("s") * 2 + lax.axis_index("c")
    base = wid * _IDX_PER_W
    pltpu.sync_copy(idx_hbm.at[pl.ds(base, _IDX_PER_W)], idx_v)
    pltpu.async_copy(table_hbm.at[idx_v], rows_v, sem).wait()
    pltpu.sync_copy(rows_v, out_hbm.at[pl.ds(base, _IDX_PER_W)])


def _sc_gather(role_table, idx_flat):
    mesh = plsc.VectorSubcoreMesh(core_axis_name="c", subcore_axis_name="s")
    f = pl.kernel(
        _sc_gather_body,
        out_type=jax.ShapeDtypeStruct((_QPAD, _RD), jnp.float32),
        mesh=mesh,
        scratch_types=[
            pltpu.VMEM((_IDX_PER_W,), jnp.int32),
            pltpu.VMEM((_IDX_PER_W, _RD), jnp.float32),
            pltpu.SemaphoreType.DMA,
        ],
        compiler_params=pltpu.CompilerParams(use_tc_tiling_on_sc=False),
    )
    return f(role_table, idx_flat)


def _tc_body(emb_ref, h_ref, filler_ref, out_ref, fg_ref, dscr_ref, fnscr_ref,
             bestv_ref, besti_ref):
    k = pl.program_id(0)
    nsteps = pl.num_programs(0) - 1

    @pl.when(k == 0)
    def _init():
        for b in range(_B):
            e_b = emb_ref[pl.ds(_S * b, _S), :]
            h_b = h_ref[b]
            fgp = lax.dot_general(
                e_b, h_b, (((1,), (1,)), ((), ())),
                preferred_element_type=jnp.float32,
                precision=lax.Precision.DEFAULT,
            )
            fg_ref[pl.ds(_S * b, _S), :] = -2.0 * fgp
        bestv_ref[...] = jnp.full((1, _Q), jnp.inf, jnp.float32)
        besti_ref[...] = jnp.zeros((1, _Q), jnp.float32)

    @pl.when(k < nsteps)
    def _produce():
        blk = filler_ref[...]
        dscr_ref[k % 2] = lax.dot_general(
            blk, fg_ref[...], (((1,), (1,)), ((), ())),
            preferred_element_type=jnp.float32,
            precision=lax.Precision.DEFAULT,
        )
        fnscr_ref[k % 2] = jnp.sum(blk * blk, axis=1, keepdims=True)

    @pl.when(k > 0)
    def _consume():
        kb = (k - 1) % 2
        sc = dscr_ref[kb] + fnscr_ref[kb]
        m = jnp.min(sc, axis=0, keepdims=True)
        idx = jnp.argmin(sc, axis=0).astype(jnp.float32)[None, :]
        idx = idx + (k - 1).astype(jnp.float32) * jnp.float32(_CHUNK)
        better = m < bestv_ref[...]
        besti_ref[...] = jnp.where(better, idx, besti_ref[...])
        bestv_ref[...] = jnp.where(better, m, bestv_ref[...])

    @pl.when(k == nsteps)
    def _fin():
        out_ref[...] = besti_ref[...].astype(jnp.int32)


def _tc_call(roles_emb, h, filler_table, interpret=False):
    nsteps = _NF // _CHUNK
    return pl.pallas_call(
        _tc_body,
        grid=(nsteps + 1,),
        in_specs=[
            pl.BlockSpec((_Q, _RD), lambda k: (0, 0)),
            pl.BlockSpec((_B, _FD, _RD), lambda k: (0, 0, 0)),
            pl.BlockSpec((_CHUNK, _FD),
                         lambda k: (jnp.minimum(k, _NF // _CHUNK - 1), 0)),
        ],
        out_specs=pl.BlockSpec((1, _Q), lambda k: (0, 0)),
        out_shape=jax.ShapeDtypeStruct((1, _Q), jnp.int32),
        scratch_shapes=[
            pltpu.VMEM((_Q, _RD), jnp.float32),
            pltpu.VMEM((2, _CHUNK, _Q), jnp.float32),
            pltpu.VMEM((2, _CHUNK, 1), jnp.float32),
            pltpu.VMEM((1, _Q), jnp.float32),
            pltpu.VMEM((1, _Q), jnp.float32),
        ],
        interpret=interpret,
    )(roles_emb, h, filler_table)


@jax.jit
def kernel(roles, hidden, filler_table, role_table):
    idx_flat = jnp.pad(roles.reshape(-1).astype(jnp.int32), (0, _QPAD - _Q))
    roles_emb = _sc_gather(role_table, idx_flat)[:_Q]
    h = hidden.reshape(_B, _FD, _RD)
    preds = _tc_call(roles_emb, h, filler_table)
    return preds.reshape(_B, _S)


# speedup vs baseline: 1.1823x; 1.1823x over previous
import functools

import jax
import jax.numpy as jnp
from jax import lax
from jax.experimental import pallas as pl
from jax.experimental.pallas import tpu as pltpu
from jax.experimental.pallas import tpu_sc as plsc

_B, _S = 16, 20
_FD, _RD = 32, 32
_NF, _NR = 100000, 200
_Q = _B * _S
_QPAD = 512
_IDX_PER_W = _QPAD // 32
_CHUNK = 10000


def _sc_gather_body(table_hbm, idx_hbm, out_hbm, idx_v, rows_v, sem):
    wid = lax.axis_index("s") * 2 + lax.axis_index("c")
    base = wid * _IDX_PER_W
    pltpu.sync_copy(idx_hbm.at[pl.ds(base, _IDX_PER_W)], idx_v)
    pltpu.async_copy(table_hbm.at[idx_v], rows_v, sem).wait()
    pltpu.sync_copy(rows_v, out_hbm.at[pl.ds(base, _IDX_PER_W)])


def _sc_gather(role_table, idx_flat):
    mesh = plsc.VectorSubcoreMesh(core_axis_name="c", subcore_axis_name="s")
    f = pl.kernel(
        _sc_gather_body,
        out_type=jax.ShapeDtypeStruct((_QPAD, _RD), jnp.float32),
        mesh=mesh,
        scratch_types=[
            pltpu.VMEM((_IDX_PER_W,), jnp.int32),
            pltpu.VMEM((_IDX_PER_W, _RD), jnp.float32),
            pltpu.SemaphoreType.DMA,
        ],
        compiler_params=pltpu.CompilerParams(use_tc_tiling_on_sc=False),
    )
    return f(role_table, idx_flat)


def _tc_body(emb_ref, h_ref, filler_ref, out_ref, fg_ref, bestv_ref, besti_ref):
    k = pl.program_id(0)

    @pl.when(k == 0)
    def _init():
        for b in range(_B):
            e_b = emb_ref[pl.ds(_S * b, _S), :]
            h_b = h_ref[b]
            fgp = lax.dot_general(
                e_b, h_b, (((1,), (1,)), ((), ())),
                preferred_element_type=jnp.float32,
                precision=lax.Precision.DEFAULT,
            )
            fg_ref[pl.ds(_S * b, _S), :] = -2.0 * fgp
        bestv_ref[...] = jnp.full((1, _Q), jnp.inf, jnp.float32)
        besti_ref[...] = jnp.zeros((1, _Q), jnp.float32)

    blk = filler_ref[...]
    d = lax.dot_general(
        blk, fg_ref[...], (((1,), (1,)), ((), ())),
        preferred_element_type=jnp.float32,
        precision=lax.Precision.DEFAULT,
    )
    fn = jnp.sum(blk * blk, axis=1, keepdims=True)
    sc = d + fn
    m = jnp.min(sc, axis=0, keepdims=True)
    idx = jnp.argmin(sc, axis=0).astype(jnp.float32)[None, :]
    idx = idx + jnp.float32(k * _CHUNK)
    better = m < bestv_ref[...]
    besti_ref[...] = jnp.where(better, idx, besti_ref[...])
    bestv_ref[...] = jnp.where(better, m, bestv_ref[...])

    @pl.when(k == pl.num_programs(0) - 1)
    def _fin():
        out_ref[...] = besti_ref[...].astype(jnp.int32)


def _tc_call(roles_emb, h, filler_table, interpret=False):
    grid = _NF // _CHUNK
    return pl.pallas_call(
        _tc_body,
        grid=(grid,),
        in_specs=[
            pl.BlockSpec((_Q, _RD), lambda k: (0, 0)),
            pl.BlockSpec((_B, _FD, _RD), lambda k: (0, 0, 0)),
            pl.BlockSpec((_CHUNK, _FD), lambda k: (k, 0)),
        ],
        out_specs=pl.BlockSpec((1, _Q), lambda k: (0, 0)),
        out_shape=jax.ShapeDtypeStruct((1, _Q), jnp.int32),
        scratch_shapes=[
            pltpu.VMEM((_Q, _RD), jnp.float32),
            pltpu.VMEM((1, _Q), jnp.float32),
            pltpu.VMEM((1, _Q), jnp.float32),
        ],
        interpret=interpret,
    )(roles_emb, h, filler_table)


@jax.jit
def kernel(roles, hidden, filler_table, role_table):
    idx_flat = jnp.pad(roles.reshape(-1).astype(jnp.int32), (0, _QPAD - _Q))
    roles_emb = _sc_gather(role_table, idx_flat)[:_Q]
    h = hidden.reshape(_B, _FD, _RD)
    preds = _tc_call(roles_emb, h, filler_table)
    return preds.reshape(_B, _S)
